# baseline (speedup 1.0000x reference)
import jax
import jax.numpy as jnp
from jax import lax
from jax.experimental import pallas as pl
from jax.experimental.pallas import tpu as pltpu
from jax.experimental.pallas import tpu_sc as plsc

B, S, D = 4, 8192, 1024
NW = 32
RPW = S // NW
R = 8
CH = R * D
NCH = RPW // R
L = 16
UNROLL = 8


def _body(x_hbm, pe_hbm, out_hbm, xbufs, obufs, pebufs, in_sem, out_sem, pe_sem):
    wid = lax.axis_index("s") * 2 + lax.axis_index("c")
    row0 = wid * RPW

    def pe_off(g):
        return (row0 + g * R) * D

    def x_off(g, b):
        return (b * S + row0 + g * R) * D

    def issue_pe(g, ph):
        pltpu.async_copy(pe_hbm.at[pl.ds(pe_off(g), CH)], pebufs.at[ph], pe_sem)

    def drain_pe(ph):
        pltpu.make_async_copy(pe_hbm.at[pl.ds(0, CH)], pebufs.at[ph], pe_sem).wait()

    def issue_in(g, b):
        pltpu.async_copy(x_hbm.at[pl.ds(x_off(g, b), CH)], xbufs.at[b], in_sem)

    def drain_in(b):
        pltpu.make_async_copy(x_hbm.at[pl.ds(0, CH)], xbufs.at[b], in_sem).wait()

    def issue_out(g, b):
        pltpu.async_copy(obufs.at[b], out_hbm.at[pl.ds(x_off(g, b), CH)], out_sem)

    def drain_out(b):
        pltpu.make_async_copy(obufs.at[b], out_hbm.at[pl.ds(0, CH)], out_sem).wait()

    def compute(b, ph):
        def add_body(i, _):
            base = i * (L * UNROLL)
            for k in range(UNROLL):
                sl = pl.ds(base + k * L, L)
                obufs[b, sl] = xbufs[b, sl] + pebufs[ph, sl]
            return 0

        lax.fori_loop(0, CH // (L * UNROLL), add_body, 0)

    def chunk(g, ph, first=False, last=False):
        drain_pe(ph)
        if not last:
            issue_pe(g + 1, 1 - ph)
        for b in range(B):
            drain_in(b)
            if not first:
                drain_out(b)
            compute(b, ph)
            issue_out(g, b)
            if not last:
                issue_in(g + 1, b)

    issue_pe(0, 0)
    for b in range(B):
        issue_in(0, b)
    chunk(0, 0, first=True)
    chunk(1, 1)

    def pair_body(g2, _):
        g = g2 * 2
        chunk(g, 0)
        chunk(g + 1, 1)
        return 0

    lax.fori_loop(1, NCH // 2 - 1, pair_body, 0)

    chunk(NCH - 2, 0)
    chunk(NCH - 1, 1, last=True)
    for b in range(B):
        drain_out(b)


def kernel(x, pe_table):
    mesh = plsc.VectorSubcoreMesh(core_axis_name="c", subcore_axis_name="s")
    run = pl.kernel(
        _body,
        mesh=mesh,
        out_type=jax.ShapeDtypeStruct((B * S * D,), jnp.float32),
        scratch_types=[
            pltpu.VMEM((B, CH), jnp.float32),
            pltpu.VMEM((B, CH), jnp.float32),
            pltpu.VMEM((2, CH), jnp.float32),
            pltpu.SemaphoreType.DMA,
            pltpu.SemaphoreType.DMA,
            pltpu.SemaphoreType.DMA,
        ],
    )
    out = run(x.reshape(B * S * D), pe_table.reshape(S * D))
    return out.reshape(B, S, D)


# speedup vs baseline: 1.0036x; 1.0036x over previous
import jax
import jax.numpy as jnp
from jax import lax
from jax.experimental import pallas as pl
from jax.experimental.pallas import tpu as pltpu
from jax.experimental.pallas import tpu_sc as plsc

B, S, D = 4, 8192, 1024
NW = 32
RPW = S // NW
R = 8
CH = R * D
NCH = RPW // R
NPAIR = NCH // 2
L = 16
UNROLL = 16


def _body(x_hbm, pe_hbm, out_hbm, xbufs, obufs, pebufs, in_sem, out_sem, pe_sem):
    wid = lax.axis_index("s") * 2 + lax.axis_index("c")
    row0 = wid * RPW

    def pe_off(g):
        return (row0 + g * R) * D

    def x_off(g, b):
        return (b * S + row0 + g * R) * D

    def issue_pe(g, ph):
        pltpu.async_copy(pe_hbm.at[pl.ds(pe_off(g), CH)], pebufs.at[ph], pe_sem)

    def drain_pe(ph):
        pltpu.make_async_copy(pe_hbm.at[pl.ds(0, CH)], pebufs.at[ph], pe_sem).wait()

    def issue_in(g, b):
        pltpu.async_copy(x_hbm.at[pl.ds(x_off(g, b), CH)], xbufs.at[b], in_sem)

    def drain_in(b):
        pltpu.make_async_copy(x_hbm.at[pl.ds(0, CH)], xbufs.at[b], in_sem).wait()

    def issue_out(g, b):
        pltpu.async_copy(obufs.at[b], out_hbm.at[pl.ds(x_off(g, b), CH)], out_sem)

    def drain_out(b):
        pltpu.make_async_copy(obufs.at[b], out_hbm.at[pl.ds(0, CH)], out_sem).wait()

    def compute(b, ph):
        def add_body(i, _):
            base = i * (L * UNROLL)
            for k in range(UNROLL):
                sl = pl.ds(base + k * L, L)
                obufs[b, sl] = xbufs[b, sl] + pebufs[ph, sl]
            return 0

        lax.fori_loop(0, CH // (L * UNROLL), add_body, 0)

    issue_pe(0, 0)
    for b in range(B):
        issue_in(0, b)

    def pair_body(g2, _):
        g = g2 * 2
        drain_pe(0)
        issue_pe(g + 1, 1)
        for b in range(B):
            drain_in(b)

            @pl.when(g2 > 0)
            def _():
                drain_out(b)

            compute(b, 0)
            issue_out(g, b)
            issue_in(g + 1, b)

        drain_pe(1)

        @pl.when(g2 < NPAIR - 1)
        def _():
            issue_pe(g + 2, 0)

        for b in range(B):
            drain_in(b)
            drain_out(b)
            compute(b, 1)
            issue_out(g + 1, b)

            @pl.when(g2 < NPAIR - 1)
            def _():
                issue_in(g + 2, b)

        return 0

    lax.fori_loop(0, NPAIR, pair_body, 0)

    for b in range(B):
        drain_out(b)


def kernel(x, pe_table):
    mesh = plsc.VectorSubcoreMesh(core_axis_name="c", subcore_axis_name="s")
    run = pl.kernel(
        _body,
        mesh=mesh,
        out_type=jax.ShapeDtypeStruct((B * S * D,), jnp.float32),
        scratch_types=[
            pltpu.VMEM((B, CH), jnp.float32),
            pltpu.VMEM((B, CH), jnp.float32),
            pltpu.VMEM((2, CH), jnp.float32),
            pltpu.SemaphoreType.DMA,
            pltpu.SemaphoreType.DMA,
            pltpu.SemaphoreType.DMA,
        ],
    )
    out = run(x.reshape(B * S * D), pe_table.reshape(S * D))
    return out.reshape(B, S, D)


# speedup vs baseline: 3.3959x; 3.3837x over previous
import jax
import jax.numpy as jnp
from jax import lax
from jax.experimental import pallas as pl
from jax.experimental.pallas import tpu as pltpu
from jax.experimental.pallas import tpu_sc as plsc

B, S, D = 4, 8192, 1024
NW = 32
RPW = S // NW
R = 8
NCH = RPW // R
NPAIR = NCH // 2
L = 16
UNROLL = 16
NVEC = R * D // L


def _body(x_hbm, pe_hbm, out_hbm, xin, obuf, pebufs, in_sem, out_sem, pe_sem):
    wid = lax.axis_index("s") * 2 + lax.axis_index("c")
    row0 = wid * RPW

    def issue_pe(g, ph):
        pltpu.async_copy(
            pe_hbm.at[pl.ds(row0 + g * R, R), :], pebufs.at[ph], pe_sem
        )

    def drain_pe(ph):
        pltpu.make_async_copy(
            pe_hbm.at[pl.ds(0, R), :], pebufs.at[ph], pe_sem
        ).wait()

    def issue_in(g, ph):
        pltpu.async_copy(
            x_hbm.at[:, pl.ds(row0 + g * R, R), :], xin.at[ph], in_sem
        )

    def drain_in(ph):
        pltpu.make_async_copy(
            x_hbm.at[:, pl.ds(0, R), :], xin.at[ph], in_sem
        ).wait()

    def issue_out(g):
        pltpu.async_copy(
            obuf, out_hbm.at[:, pl.ds(row0 + g * R, R), :], out_sem
        )

    def drain_out():
        pltpu.make_async_copy(
            obuf, out_hbm.at[:, pl.ds(0, R), :], out_sem
        ).wait()

    def compute(ph):
        for b in range(B):

            def add_body(i, _):
                for k in range(UNROLL):
                    flat = i * UNROLL + k
                    r = flat // (D // L)
                    c = (flat % (D // L)) * L
                    sl = pl.ds(c, L)
                    obuf[b, r, sl] = xin[ph, b, r, sl] + pebufs[ph, r, sl]
                return 0

            lax.fori_loop(0, NVEC // UNROLL, add_body, 0)

    def _maybe(cond, fn):
        if cond is True:
            fn()
        else:
            pl.when(cond)(fn)

    def chunk(g, g2, ph, issue_more):
        drain_in(ph)
        _maybe(issue_more, lambda: issue_in(g + 1, 1 - ph))
        drain_pe(ph)
        _maybe(issue_more, lambda: issue_pe(g + 1, 1 - ph))
        _maybe(True if ph > 0 else g2 > 0, drain_out)
        compute(ph)
        issue_out(g)

    issue_in(0, 0)
    issue_pe(0, 0)

    def pair_body(g2, _):
        g = g2 * 2
        chunk(g, g2, 0, True)
        chunk(g + 1, g2, 1, g2 < NPAIR - 1)
        return 0

    lax.fori_loop(0, NPAIR, pair_body, 0)
    drain_out()


def kernel(x, pe_table):
    mesh = plsc.VectorSubcoreMesh(core_axis_name="c", subcore_axis_name="s")
    run = pl.kernel(
        _body,
        mesh=mesh,
        out_type=jax.ShapeDtypeStruct((B, S, D), jnp.float32),
        scratch_types=[
            pltpu.VMEM((2, B, R, D), jnp.float32),
            pltpu.VMEM((B, R, D), jnp.float32),
            pltpu.VMEM((2, R, D), jnp.float32),
            pltpu.SemaphoreType.DMA,
            pltpu.SemaphoreType.DMA,
            pltpu.SemaphoreType.DMA,
        ],
        compiler_params=pltpu.CompilerParams(use_tc_tiling_on_sc=True),
    )
    return run(x, pe_table)
